# baseline (speedup 1.0000x reference)
import functools
import math

import jax
import jax.numpy as jnp
from jax import lax
from jax.experimental import pallas as pl
from jax.experimental.pallas import tpu as pltpu
from jax.experimental.pallas import tpu_sc as plsc

_MIN_KEPT = 100000
_THRESH = 0.7
_TAU0 = float(-math.log(_THRESH))

_BH = 256
_NC = 2
_NT = 16
_LN = 16


def _ce_body(pred_ref, tgt_ref, out_ref):
    x = pred_ref[0]
    t = tgt_ref[0]
    m = jnp.max(x, axis=0)
    s = jnp.sum(jnp.exp(x - m[None]), axis=0)
    cls = lax.broadcasted_iota(jnp.int32, x.shape, 0)
    xt = jnp.sum(jnp.where(cls == t[None], x, 0.0), axis=0)
    out_ref[0] = ((m - xt) + jnp.log(s)).astype(jnp.bfloat16)


def _ce_losses(predict, target, b0, nb):
    _, C, H, W = predict.shape
    return pl.pallas_call(
        _ce_body,
        grid=(nb, H // _BH),
        in_specs=[
            pl.BlockSpec((1, C, _BH, W), lambda b, h: (b + b0, 0, h, 0)),
            pl.BlockSpec((1, _BH, W), lambda b, h: (b + b0, h, 0)),
        ],
        out_specs=pl.BlockSpec((1, _BH, W), lambda b, h: (b, h, 0)),
        out_shape=jax.ShapeDtypeStruct((nb, H, W), jnp.bfloat16),
    )(predict, target)


@functools.lru_cache(maxsize=None)
def _make_phase1(n):
    nw = _NC * _NT
    chunk = n // nw
    iters = chunk // _LN
    mesh = plsc.VectorSubcoreMesh(
        core_axis_name="c", subcore_axis_name="s", num_cores=_NC)

    nword = n // 2
    chunk = nword // nw
    half = chunk // 2
    unroll = 2
    jblk = unroll * _LN

    @functools.partial(
        pl.kernel,
        out_type=jax.ShapeDtypeStruct((nw, 48), jnp.float32),
        mesh=mesh,
        compiler_params=pltpu.CompilerParams(needs_layout_passes=False),
        scratch_types=[
            pltpu.VMEM((chunk,), jnp.float32),
            pltpu.VMEM((48,), jnp.float32),
            pltpu.SemaphoreType.DMA,
            pltpu.SemaphoreType.DMA,
        ],
    )
    def phase1(l_hbm, out_hbm, buf, pub, sem0, sem1):
        wid = lax.axis_index("s") * _NC + lax.axis_index("c")
        zeros = jnp.zeros((_LN,), jnp.float32)
        base = wid * chunk

        cp0 = pltpu.async_copy(
            l_hbm.at[pl.ds(base, half)], buf.at[pl.ds(0, half)], sem0)
        cp1 = pltpu.async_copy(
            l_hbm.at[pl.ds(base + half, half)], buf.at[pl.ds(half, half)],
            sem1)

        def body(j, carry):
            accs = list(carry)
            j0 = pl.multiple_of(j * jblk, jblk)
            for k in range(unroll):
                g, e, s = accs[3 * k:3 * k + 3]
                vw = buf[pl.ds(j0 + k * _LN, _LN)]
                vp = plsc.bitcast(vw, jnp.bfloat16)
                va, vb = plsc.unpack(vp, format=plsc.PackFormat.INTERLEAVED)
                for v in (va, vb):
                    g = g + jnp.where(v > _TAU0, 1.0, 0.0)
                    e = e + jnp.where(v >= _TAU0, 1.0, 0.0)
                    s = s + jnp.where(v > _TAU0, v, 0.0)
                accs[3 * k:3 * k + 3] = [g, e, s]
            return tuple(accs)

        carry = (zeros,) * (3 * unroll)
        cp0.wait()
        carry = lax.fori_loop(0, half // jblk, body, carry)
        cp1.wait()
        carry = lax.fori_loop(half // jblk, chunk // jblk, body, carry)

        g = carry[0] + carry[3]
        e = carry[1] + carry[4]
        s = carry[2] + carry[5]
        pub[pl.ds(0, _LN)] = g
        pub[pl.ds(16, _LN)] = e
        pub[pl.ds(32, _LN)] = s
        pltpu.sync_copy(pub, out_hbm.at[wid])

    return phase1


@functools.lru_cache(maxsize=None)
def _make_fallback(n):
    chunk = (n // 2) // _NT
    iters = chunk // _LN
    kept = min(_MIN_KEPT, n - 1)
    rank = float(n - 1 - kept)
    mesh = plsc.VectorSubcoreMesh(
        core_axis_name="c", subcore_axis_name="s", num_cores=1)

    @functools.partial(
        pl.kernel,
        out_type=jax.ShapeDtypeStruct((_LN,), jnp.float32),
        mesh=mesh,
        compiler_params=pltpu.CompilerParams(needs_layout_passes=False),
        scratch_types=[
            pltpu.VMEM((chunk,), jnp.float32),
            pltpu.VMEM_SHARED((_NT * 16,), jnp.float32),
            pltpu.VMEM((_NT * 16,), jnp.float32),
            pltpu.VMEM((_LN,), jnp.float32),
            pltpu.VMEM((_LN,), jnp.float32),
        ],
    )
    def fb(l_hbm, out_hbm, buf, stage, stage_l, pub, obuf):
        wid = lax.axis_index("s")
        zeros = jnp.zeros((_LN,), jnp.float32)
        lane = lax.broadcasted_iota(jnp.int32, (_LN,), 0)

        pltpu.sync_copy(l_hbm.at[pl.ds(wid * chunk, chunk)], buf)

        def vchunk(j):
            vw = buf[pl.ds(pl.multiple_of(j * _LN, _LN), _LN)]
            vp = plsc.bitcast(vw, jnp.bfloat16)
            return plsc.unpack(vp, format=plsc.PackFormat.INTERLEAVED)

        def vec_to_scalar(v):
            acc = v[0]
            for i in range(1, _LN):
                acc = acc + v[i]
            return acc

        def combine(a):
            pub[pl.ds(0, _LN)] = a
            pltpu.sync_copy(pub, stage.at[pl.ds(wid * 16, _LN)])
            plsc.subcore_barrier()
            pltpu.sync_copy(stage, stage_l)
            ta = zeros
            for t in range(_NT):
                ta = ta + stage_l[pl.ds(t * 16, _LN)]
            plsc.subcore_barrier()
            return vec_to_scalar(ta)

        def bit_round(i, p):
            t_pat = p | lax.shift_left(jnp.int32(1), jnp.int32(30) - i)
            t_val = lax.bitcast_convert_type(t_pat, jnp.float32)

            def cbody(j, acc):
                va, vb = vchunk(j)
                acc = acc + jnp.where(va < t_val, 1.0, 0.0)
                return acc + jnp.where(vb < t_val, 1.0, 0.0)

            cl = lax.fori_loop(0, iters, cbody, zeros)
            total = combine(cl)
            return jnp.where(total <= rank, t_pat, p)

        p = lax.fori_loop(0, 31, bit_round, jnp.int32(0))
        cutoff = lax.bitcast_convert_type(p, jnp.float32)

        def fbody(j, carry):
            g2, s2 = carry
            for v in vchunk(j):
                keep = v > cutoff
                g2 = g2 + jnp.where(keep, 1.0, 0.0)
                s2 = s2 + jnp.where(keep, v, 0.0)
            return g2, s2

        g2, s2 = lax.fori_loop(0, iters, fbody, (zeros, zeros))
        c_d = combine(g2)
        s_d = combine(s2)

        @pl.when(wid == 0)
        def _():
            obuf[...] = jnp.where(
                lane == 0, s_d, jnp.where(lane == 1, c_d, 0.0))
            pltpu.sync_copy(obuf, out_hbm)

    return fb


def kernel(predict, target):
    target = target.astype(jnp.int32)
    B = predict.shape[0]
    ce = _ce_losses(predict, target, 0, B)
    n = ce.size
    flat = lax.bitcast_convert_type(ce.reshape(n // 2, 2), jnp.float32)
    parts = _make_phase1(n)(flat)
    c_gt = jnp.sum(parts[:, 0:16])
    c_ge = jnp.sum(parts[:, 16:32])
    s_gt = jnp.sum(parts[:, 32:48])
    kept_f = jnp.float32(min(_MIN_KEPT, n - 1))

    def rare(_):
        out = _make_fallback(n)(flat)
        return out[0], out[1]

    s_sel, c_sel = lax.cond(
        c_ge <= kept_f, rare, lambda _: (s_gt, c_gt), None)
    return jnp.where(c_sel > 0.0, s_sel / jnp.maximum(c_sel, 1.0), 0.0)


# speedup vs baseline: 6.5976x; 6.5976x over previous
import functools
import math

import jax
import jax.numpy as jnp
from jax import lax
from jax.experimental import pallas as pl
from jax.experimental.pallas import tpu as pltpu
from jax.experimental.pallas import tpu_sc as plsc

_MIN_KEPT = 100000
_THRESH = 0.7
_TAU0 = float(-math.log(_THRESH))

_BH = 256
_NC = 2
_NT = 16
_LN = 16


def _ce_body(pred_ref, tgt_ref, out_ref):
    x = pred_ref[0]
    t = tgt_ref[0]
    m = jnp.max(x, axis=0)
    s = jnp.sum(jnp.exp(x - m[None]), axis=0)
    cls = lax.broadcasted_iota(jnp.int32, x.shape, 0)
    xt = jnp.sum(jnp.where(cls == t[None], x, 0.0), axis=0)
    ce = (m - xt) + jnp.log(s)
    w = ce.shape[-1] // 2

    def rnd(v):
        u = lax.bitcast_convert_type(v, jnp.uint32)
        return u + jnp.uint32(0x7FFF) + ((u >> 16) & jnp.uint32(1))

    wa = rnd(ce[:, :w]) >> 16
    wb = rnd(ce[:, w:]) & jnp.uint32(0xFFFF0000)
    out_ref[0] = lax.bitcast_convert_type(wa | wb, jnp.float32)


def _ce_losses(predict, target, b0, nb):
    _, C, H, W = predict.shape
    return pl.pallas_call(
        _ce_body,
        grid=(nb, H // _BH),
        in_specs=[
            pl.BlockSpec((1, C, _BH, W), lambda b, h: (b + b0, 0, h, 0)),
            pl.BlockSpec((1, _BH, W), lambda b, h: (b + b0, h, 0)),
        ],
        out_specs=pl.BlockSpec((1, _BH, W // 2), lambda b, h: (b, h, 0)),
        out_shape=jax.ShapeDtypeStruct((nb, H, W // 2), jnp.float32),
    )(predict, target)


@functools.lru_cache(maxsize=None)
def _make_phase1(n):
    nw = _NC * _NT
    chunk = n // nw
    iters = chunk // _LN
    mesh = plsc.VectorSubcoreMesh(
        core_axis_name="c", subcore_axis_name="s", num_cores=_NC)

    nword = n // 2
    chunk = nword // nw
    half = chunk // 2
    unroll = 2
    jblk = unroll * _LN

    @functools.partial(
        pl.kernel,
        out_type=jax.ShapeDtypeStruct((nw, 48), jnp.float32),
        mesh=mesh,
        compiler_params=pltpu.CompilerParams(needs_layout_passes=False),
        scratch_types=[
            pltpu.VMEM((chunk,), jnp.float32),
            pltpu.VMEM((48,), jnp.float32),
            pltpu.SemaphoreType.DMA,
            pltpu.SemaphoreType.DMA,
        ],
    )
    def phase1(l_hbm, out_hbm, buf, pub, sem0, sem1):
        wid = lax.axis_index("s") * _NC + lax.axis_index("c")
        zeros = jnp.zeros((_LN,), jnp.float32)
        base = wid * chunk

        cp0 = pltpu.async_copy(
            l_hbm.at[pl.ds(base, half)], buf.at[pl.ds(0, half)], sem0)
        cp1 = pltpu.async_copy(
            l_hbm.at[pl.ds(base + half, half)], buf.at[pl.ds(half, half)],
            sem1)

        def body(j, carry):
            accs = list(carry)
            j0 = pl.multiple_of(j * jblk, jblk)
            for k in range(unroll):
                g, e, s = accs[3 * k:3 * k + 3]
                vw = buf[pl.ds(j0 + k * _LN, _LN)]
                vp = plsc.bitcast(vw, jnp.bfloat16)
                va, vb = plsc.unpack(vp, format=plsc.PackFormat.INTERLEAVED)
                for v in (va, vb):
                    g = g + jnp.where(v > _TAU0, 1.0, 0.0)
                    e = e + jnp.where(v >= _TAU0, 1.0, 0.0)
                    s = s + jnp.where(v > _TAU0, v, 0.0)
                accs[3 * k:3 * k + 3] = [g, e, s]
            return tuple(accs)

        carry = (zeros,) * (3 * unroll)
        cp0.wait()
        carry = lax.fori_loop(0, half // jblk, body, carry)
        cp1.wait()
        carry = lax.fori_loop(half // jblk, chunk // jblk, body, carry)

        g = carry[0] + carry[3]
        e = carry[1] + carry[4]
        s = carry[2] + carry[5]
        pub[pl.ds(0, _LN)] = g
        pub[pl.ds(16, _LN)] = e
        pub[pl.ds(32, _LN)] = s
        pltpu.sync_copy(pub, out_hbm.at[wid])

    return phase1


@functools.lru_cache(maxsize=None)
def _make_fallback(n):
    chunk = (n // 2) // _NT
    iters = chunk // _LN
    kept = min(_MIN_KEPT, n - 1)
    rank = float(n - 1 - kept)
    mesh = plsc.VectorSubcoreMesh(
        core_axis_name="c", subcore_axis_name="s", num_cores=1)

    @functools.partial(
        pl.kernel,
        out_type=jax.ShapeDtypeStruct((_LN,), jnp.float32),
        mesh=mesh,
        compiler_params=pltpu.CompilerParams(needs_layout_passes=False),
        scratch_types=[
            pltpu.VMEM((chunk,), jnp.float32),
            pltpu.VMEM_SHARED((_NT * 16,), jnp.float32),
            pltpu.VMEM((_NT * 16,), jnp.float32),
            pltpu.VMEM((_LN,), jnp.float32),
            pltpu.VMEM((_LN,), jnp.float32),
        ],
    )
    def fb(l_hbm, out_hbm, buf, stage, stage_l, pub, obuf):
        wid = lax.axis_index("s")
        zeros = jnp.zeros((_LN,), jnp.float32)
        lane = lax.broadcasted_iota(jnp.int32, (_LN,), 0)

        pltpu.sync_copy(l_hbm.at[pl.ds(wid * chunk, chunk)], buf)

        def vchunk(j):
            vw = buf[pl.ds(pl.multiple_of(j * _LN, _LN), _LN)]
            vp = plsc.bitcast(vw, jnp.bfloat16)
            return plsc.unpack(vp, format=plsc.PackFormat.INTERLEAVED)

        def vec_to_scalar(v):
            acc = v[0]
            for i in range(1, _LN):
                acc = acc + v[i]
            return acc

        def combine(a):
            pub[pl.ds(0, _LN)] = a
            pltpu.sync_copy(pub, stage.at[pl.ds(wid * 16, _LN)])
            plsc.subcore_barrier()
            pltpu.sync_copy(stage, stage_l)
            ta = zeros
            for t in range(_NT):
                ta = ta + stage_l[pl.ds(t * 16, _LN)]
            plsc.subcore_barrier()
            return vec_to_scalar(ta)

        def bit_round(i, p):
            t_pat = p | lax.shift_left(jnp.int32(1), jnp.int32(30) - i)
            t_val = lax.bitcast_convert_type(t_pat, jnp.float32)

            def cbody(j, acc):
                va, vb = vchunk(j)
                acc = acc + jnp.where(va < t_val, 1.0, 0.0)
                return acc + jnp.where(vb < t_val, 1.0, 0.0)

            cl = lax.fori_loop(0, iters, cbody, zeros)
            total = combine(cl)
            return jnp.where(total <= rank, t_pat, p)

        p = lax.fori_loop(0, 31, bit_round, jnp.int32(0))
        cutoff = lax.bitcast_convert_type(p, jnp.float32)

        def fbody(j, carry):
            g2, s2 = carry
            for v in vchunk(j):
                keep = v > cutoff
                g2 = g2 + jnp.where(keep, 1.0, 0.0)
                s2 = s2 + jnp.where(keep, v, 0.0)
            return g2, s2

        g2, s2 = lax.fori_loop(0, iters, fbody, (zeros, zeros))
        c_d = combine(g2)
        s_d = combine(s2)

        @pl.when(wid == 0)
        def _():
            obuf[...] = jnp.where(
                lane == 0, s_d, jnp.where(lane == 1, c_d, 0.0))
            pltpu.sync_copy(obuf, out_hbm)

    return fb


def kernel(predict, target):
    target = target.astype(jnp.int32)
    B = predict.shape[0]
    cw = _ce_losses(predict, target, 0, B)
    flat = cw.reshape(-1)
    n = flat.shape[0] * 2
    parts = _make_phase1(n)(flat)
    c_gt = jnp.sum(parts[:, 0:16])
    c_ge = jnp.sum(parts[:, 16:32])
    s_gt = jnp.sum(parts[:, 32:48])
    kept_f = jnp.float32(min(_MIN_KEPT, n - 1))

    def rare(_):
        out = _make_fallback(n)(flat)
        return out[0], out[1]

    s_sel, c_sel = lax.cond(
        c_ge <= kept_f, rare, lambda _: (s_gt, c_gt), None)
    return jnp.where(c_sel > 0.0, s_sel / jnp.maximum(c_sel, 1.0), 0.0)
